# baseline (speedup 1.0000x reference)
import jax
import jax.numpy as jnp
from jax.experimental import pallas as pl

TOPK = 8
SEQ = 8192
FEAT = 128
CHUNK = 8
_NEG = float("-inf")

_NET8 = (
    (0, 1), (2, 3), (4, 5), (6, 7),
    (0, 2), (1, 3), (4, 6), (5, 7),
    (1, 2), (5, 6), (0, 4), (3, 7),
    (1, 5), (2, 6),
    (1, 4), (3, 6),
    (2, 4), (3, 5),
    (3, 4),
)


def _sort8(v):
    v = list(v)
    for a, b in _NET8:
        hi = jnp.maximum(v[a], v[b])
        lo = jnp.minimum(v[a], v[b])
        v[a], v[b] = hi, lo
    return v


def _merge_top8(a, b):
    h = [jnp.maximum(a[j], b[7 - j]) for j in range(8)]
    for d in (4, 2, 1):
        nh = list(h)
        for s in range(0, 8, 2 * d):
            for t in range(s, s + d):
                nh[t] = jnp.maximum(h[t], h[t + d])
                nh[t + d] = jnp.minimum(h[t], h[t + d])
        h = nh
    return h


def _kmax_body(x_ref, o_ref):
    group = 8 * CHUNK

    def body(i, acc):
        blk = x_ref[0, pl.ds(i * 4 * group, 4 * group), :]
        v = [_sort8(blk[(8 * g + j) * CHUNK:(8 * g + j + 1) * CHUNK, :]
                    for j in range(8)) for g in range(4)]
        w = _merge_top8(_merge_top8(v[0], v[1]), _merge_top8(v[2], v[3]))
        return tuple(_merge_top8(list(acc), w))

    init = tuple(jnp.full((CHUNK, FEAT), _NEG, jnp.float32) for _ in range(TOPK))
    acc = list(jax.lax.fori_loop(0, SEQ // (4 * group), body, init))

    p = CHUNK
    while p > 1:
        half = p // 2
        acc = _merge_top8([t[:half] for t in acc], [t[half:] for t in acc])
        p = half
    top = jnp.concatenate(acc, axis=0)
    o_ref[0] = top.T


def kernel(x):
    b, s, f = x.shape
    out = pl.pallas_call(
        _kmax_body,
        grid=(b,),
        in_specs=[pl.BlockSpec((1, s, f), lambda i: (i, 0, 0))],
        out_specs=pl.BlockSpec((1, f, TOPK), lambda i: (i, 0, 0)),
        out_shape=jax.ShapeDtypeStruct((b, f, TOPK), x.dtype),
    )(x)
    return out.reshape(b, f * TOPK)


# speedup vs baseline: 1.0046x; 1.0046x over previous
import jax
import jax.numpy as jnp
from jax.experimental import pallas as pl

TOPK = 8
SEQ = 8192
FEAT = 128
CHUNK = 16
_NEG = float("-inf")

_NET8 = (
    (0, 1), (2, 3), (4, 5), (6, 7),
    (0, 2), (1, 3), (4, 6), (5, 7),
    (1, 2), (5, 6), (0, 4), (3, 7),
    (1, 5), (2, 6),
    (1, 4), (3, 6),
    (2, 4), (3, 5),
    (3, 4),
)


def _sort8(v):
    v = list(v)
    for a, b in _NET8:
        hi = jnp.maximum(v[a], v[b])
        lo = jnp.minimum(v[a], v[b])
        v[a], v[b] = hi, lo
    return v


def _merge_top8(a, b):
    h = [jnp.maximum(a[j], b[7 - j]) for j in range(8)]
    for d in (4, 2, 1):
        nh = list(h)
        for s in range(0, 8, 2 * d):
            for t in range(s, s + d):
                nh[t] = jnp.maximum(h[t], h[t + d])
                nh[t + d] = jnp.minimum(h[t], h[t + d])
        h = nh
    return h


def _kmax_body(x_ref, o_ref):
    group = 8 * CHUNK

    def body(i, acc):
        blk = x_ref[0, pl.ds(i * 4 * group, 4 * group), :]
        v = [_sort8(blk[(8 * g + j) * CHUNK:(8 * g + j + 1) * CHUNK, :]
                    for j in range(8)) for g in range(4)]
        w = _merge_top8(_merge_top8(v[0], v[1]), _merge_top8(v[2], v[3]))
        return tuple(_merge_top8(list(acc), w))

    init = tuple(jnp.full((CHUNK, FEAT), _NEG, jnp.float32) for _ in range(TOPK))
    acc = list(jax.lax.fori_loop(0, SEQ // (4 * group), body, init))

    p = CHUNK
    while p > 1:
        half = p // 2
        acc = _merge_top8([t[:half] for t in acc], [t[half:] for t in acc])
        p = half
    top = jnp.concatenate(acc, axis=0)
    o_ref[0] = top.T


def kernel(x):
    b, s, f = x.shape
    out = pl.pallas_call(
        _kmax_body,
        grid=(b,),
        in_specs=[pl.BlockSpec((1, s, f), lambda i: (i, 0, 0))],
        out_specs=pl.BlockSpec((1, f, TOPK), lambda i: (i, 0, 0)),
        out_shape=jax.ShapeDtypeStruct((b, f, TOPK), x.dtype),
    )(x)
    return out.reshape(b, f * TOPK)
